# baseline (speedup 1.0000x reference)
import jax
import jax.numpy as jnp
from jax.experimental import pallas as pl
from jax.experimental.pallas import tpu as pltpu


def _fused_relu_matmul_bias(x_ref, w_ref, b_ref, o_ref):
    x = jnp.maximum(x_ref[...], 0.0)
    o_ref[...] = (
        jnp.dot(x, w_ref[...], preferred_element_type=jnp.float32) + b_ref[...]
    )


def _call(x, w, b, tile_m):
    m, d = x.shape
    n = w.shape[1]
    grid = (m // tile_m,)
    return pl.pallas_call(
        _fused_relu_matmul_bias,
        grid=grid,
        in_specs=[
            pl.BlockSpec((tile_m, d), lambda i: (i, 0)),
            pl.BlockSpec((d, n), lambda i: (0, 0)),
            pl.BlockSpec((1, n), lambda i: (0, 0)),
        ],
        out_specs=pl.BlockSpec((tile_m, n), lambda i: (i, 0)),
        out_shape=jax.ShapeDtypeStruct((m, n), jnp.float32),
        compiler_params=pltpu.CompilerParams(
            dimension_semantics=("arbitrary",),
        ),
    )(x, w, b)


def kernel(
    x_subject,
    x_region,
    edge_index_sr,
    edge_index_rr,
    edge_attr_sr,
    edge_attr_rr,
    sage_Wl0,
    sage_bl0,
    sage_Wr0,
    gcn_W0,
    gcn_b0,
    sage_Wl1,
    sage_bl1,
    sage_Wr1,
    gcn_W1,
    gcn_b1,
    lin_W,
    lin_b,
):
    m, d = x_subject.shape
    out_dim = lin_W.shape[1]

    fold = 2
    if m % fold == 0 and (fold * out_dim) % 128 == 0:
        zeros = jnp.zeros_like(lin_W)
        w2 = jnp.block([[lin_W, zeros], [zeros, lin_W]])
        b2 = jnp.concatenate([lin_b, lin_b]).reshape(1, fold * out_dim)
        x2 = x_subject.reshape(m // fold, fold * d)
        tile_m = 1000
        if (m // fold) % tile_m != 0:
            tile_m = m // fold
        out2 = _call(x2, w2, b2, tile_m)
        return out2.reshape(m, out_dim)

    bias = lin_b.reshape(1, out_dim)
    tile_m = 1000 if m % 1000 == 0 else m
    return _call(x_subject, lin_W, bias, tile_m)


# speedup vs baseline: 4.1488x; 4.1488x over previous
import jax
import jax.numpy as jnp
from jax import lax
from jax.experimental import pallas as pl
from jax.experimental.pallas import tpu as pltpu


def _fused_relu_matmul_bias_t(x_ref, wt_ref, b_ref, o_ref):
    x = jnp.maximum(x_ref[...], 0.0)
    y = (
        lax.dot_general(
            x,
            wt_ref[...],
            (((1,), (1,)), ((), ())),
            preferred_element_type=jnp.float32,
        )
        + b_ref[...]
    )
    o_ref[...] = y.T


def kernel(
    x_subject,
    x_region,
    edge_index_sr,
    edge_index_rr,
    edge_attr_sr,
    edge_attr_rr,
    sage_Wl0,
    sage_bl0,
    sage_Wr0,
    gcn_W0,
    gcn_b0,
    sage_Wl1,
    sage_bl1,
    sage_Wr1,
    gcn_W1,
    gcn_b1,
    lin_W,
    lin_b,
):
    m, d = x_subject.shape
    n = lin_W.shape[1]
    w_t = lin_W.T
    bias = lin_b.reshape(1, n)

    out_t = pl.pallas_call(
        _fused_relu_matmul_bias_t,
        grid=(1,),
        in_specs=[
            pl.BlockSpec((m, d), lambda i: (0, 0)),
            pl.BlockSpec((n, d), lambda i: (0, 0)),
            pl.BlockSpec((1, n), lambda i: (0, 0)),
        ],
        out_specs=pl.BlockSpec((n, m), lambda i: (0, 0)),
        out_shape=jax.ShapeDtypeStruct((n, m), jnp.float32),
        compiler_params=pltpu.CompilerParams(
            dimension_semantics=("arbitrary",),
        ),
    )(x_subject, w_t, bias)
    return out_t.T
